# baseline (speedup 1.0000x reference)
import functools
import jax
import jax.numpy as jnp
from jax import lax
from jax.experimental import pallas as pl
from jax.experimental.pallas import tpu as pltpu
from jax.experimental.pallas import tpu_sc as plsc

VOCAB = 100000
HIDDEN = 768
MAX_POS = 8192
TYPES = 2
EPS = 1e-12
B, S = 4, 8192
N_TOK = B * S

NC, NS, L = 2, 16, 16
NW = NC * NS
TPW = N_TOK // NW
K = 64
NCHUNK = TPW // K
NSLICE = HIDDEN // L


def _lane_bcast(v, idx):
    return lax.gather(
        v, idx[:, None],
        dimension_numbers=lax.GatherDimensionNumbers(
            offset_dims=(), collapsed_slice_dims=(0,), start_index_map=(0,)),
        slice_sizes=(1,),
        mode=lax.GatherScatterMode.PROMISE_IN_BOUNDS)


def _allreduce_sum(v):
    iota = lax.iota(jnp.int32, L)
    for sh in (8, 4, 2, 1):
        v = v + _lane_bcast(v, iota ^ sh)
    return v


def _rsqrt(v):
    bits = lax.bitcast_convert_type(v, jnp.int32)
    r = lax.bitcast_convert_type(jnp.int32(0x5F3759DF) - (bits >> 1), jnp.float32)
    for _ in range(3):
        r = r * (1.5 - 0.5 * v * r * r)
    return r


def _sc_body(ids_hbm, pos_hbm, tt_hbm, word_hbm, post_hbm, typet_hbm,
             gamma_hbm, beta_hbm, out_hbm,
             idx_w, idx_p, tt_v, w_rows, p_rows, type_v, gamma_v, beta_v,
             sem0, sem1):
    wid = lax.axis_index("s") * NC + lax.axis_index("c")

    pltpu.sync_copy(typet_hbm, type_v)
    pltpu.sync_copy(gamma_hbm, gamma_v)
    pltpu.sync_copy(beta_hbm, beta_v)

    zero16 = jnp.zeros((L,), jnp.int32)

    def chunk_body(c, carry):
        base = wid * TPW + c * K
        pltpu.sync_copy(ids_hbm.at[pl.ds(base, K)], idx_w)
        pltpu.sync_copy(pos_hbm.at[pl.ds(base, K)], idx_p)
        pltpu.sync_copy(tt_hbm.at[pl.ds(base, K)], tt_v.at[pl.ds(0, K)])
        cp_w = pltpu.async_copy(word_hbm.at[idx_w], w_rows, sem0)
        cp_p = pltpu.async_copy(post_hbm.at[idx_p], p_rows, sem1)
        cp_w.wait()
        cp_p.wait()

        def tok_body(i, carry2):
            tts = tt_v[pl.ds(i, L)]
            ttf = _lane_bcast(tts, zero16).astype(jnp.float32)

            s_acc = [jnp.zeros((L,), jnp.float32) for _ in range(4)]
            q_acc = [jnp.zeros((L,), jnp.float32) for _ in range(4)]
            for j in range(NSLICE):
                sl = pl.ds(j * L, L)
                t0 = type_v[0, sl]
                te = t0 + ttf * (type_v[1, sl] - t0)
                v = w_rows[i, sl] + p_rows[i, sl] + te
                w_rows[i, sl] = v
                s_acc[j % 4] = s_acc[j % 4] + v
                q_acc[j % 4] = q_acc[j % 4] + v * v
            s = (s_acc[0] + s_acc[1]) + (s_acc[2] + s_acc[3])
            q = (q_acc[0] + q_acc[1]) + (q_acc[2] + q_acc[3])
            mean_v = _allreduce_sum(s) * (1.0 / HIDDEN)
            var_v = _allreduce_sum(q) * (1.0 / HIDDEN) - mean_v * mean_v
            rstd = _rsqrt(var_v + EPS)

            for j in range(NSLICE):
                sl = pl.ds(j * L, L)
                v = w_rows[i, sl]
                w_rows[i, sl] = (v - mean_v) * rstd * gamma_v[sl] + beta_v[sl]
            return carry2

        lax.fori_loop(0, K, tok_body, 0, unroll=False)
        pltpu.sync_copy(w_rows, out_hbm.at[pl.ds(base, K)])
        return carry

    lax.fori_loop(0, NCHUNK, chunk_body, 0, unroll=False)


@jax.jit
def _bert_embed(ids, pos, tt, word_table, pos_table, type_table, gamma, beta):
    mesh = plsc.VectorSubcoreMesh(
        core_axis_name="c", subcore_axis_name="s", num_cores=NC, num_subcores=NS
    )
    f = pl.kernel(
        _sc_body,
        out_type=jax.ShapeDtypeStruct((N_TOK, HIDDEN), jnp.float32),
        mesh=mesh,
        scratch_types=[
            pltpu.VMEM((K,), jnp.int32),
            pltpu.VMEM((K,), jnp.int32),
            pltpu.VMEM((K + L,), jnp.int32),
            pltpu.VMEM((K, HIDDEN), jnp.float32),
            pltpu.VMEM((K, HIDDEN), jnp.float32),
            pltpu.VMEM((TYPES, HIDDEN), jnp.float32),
            pltpu.VMEM((HIDDEN,), jnp.float32),
            pltpu.VMEM((HIDDEN,), jnp.float32),
            pltpu.SemaphoreType.DMA,
            pltpu.SemaphoreType.DMA,
        ],
    )
    return f(ids, pos, tt, word_table, pos_table, type_table, gamma, beta)


def kernel(token_type_ids, position_ids, inputs_embeds, word_table, pos_table,
           type_table, gamma, beta):
    ids = inputs_embeds.reshape(N_TOK).astype(jnp.int32)
    pos = position_ids.reshape(N_TOK).astype(jnp.int32)
    tt = token_type_ids.reshape(N_TOK).astype(jnp.int32)
    out = _bert_embed(ids, pos, tt, word_table, pos_table, type_table, gamma, beta)
    return out.reshape(B, S, HIDDEN)
